# baseline (speedup 1.0000x reference)
import functools

import jax
import jax.numpy as jnp
from jax.experimental import pallas as pl

_EPS = 1e-12


def _embed_ln_kernel(ids_ref, tok_ref, tokT_ref, pos_ref, gamma_ref, beta_ref,
                     out_ref, *, vocab: int):
    b, sblk, _ = ids_ref.shape
    d = tok_ref.shape[1]
    inv_d = 1.0 / d
    tok_tab = tok_ref[...]
    tok_t = tokT_ref[...]
    pos = pos_ref[...]
    g = gamma_ref[...]
    bt = beta_ref[...]

    pos_mean = jnp.mean(pos, axis=1, keepdims=True)
    pos_sq = jnp.mean(pos * pos, axis=1, keepdims=True)
    tok_mean = jnp.mean(tok_t, axis=0, keepdims=True)
    tok_sq = jnp.mean(tok_t * tok_t, axis=0, keepdims=True)
    cross = jnp.dot(pos, tok_t, preferred_element_type=jnp.float32) * inv_d
    sv = tok_sq + 2.0 * cross

    iota = jax.lax.broadcasted_iota(jnp.int32, (sblk, vocab), 1)
    for bi in range(b):
        ids = ids_ref[bi]
        onehot = (ids == iota).astype(jnp.float32)
        m = pos_mean + jnp.sum(onehot * tok_mean, axis=1, keepdims=True)
        ex2 = pos_sq + jnp.sum(onehot * sv, axis=1, keepdims=True)
        var = ex2 - m * m
        rstd = jax.lax.rsqrt(var + _EPS)
        x = jnp.dot(onehot, tok_tab, preferred_element_type=jnp.float32) + pos
        out_ref[bi] = (x - m) * (rstd * g) + bt


def kernel(input_ids, tok_emb, pos_emb, gamma, beta):
    b, s = input_ids.shape
    vocab, d = tok_emb.shape
    sblk = 1024
    grid = (s // sblk,)

    ids = input_ids.astype(jnp.int32).reshape(b, s, 1)
    pos = pos_emb[:s]
    tok_t = tok_emb.T

    out = pl.pallas_call(
        functools.partial(_embed_ln_kernel, vocab=vocab),
        grid=grid,
        in_specs=[
            pl.BlockSpec((b, sblk, 1), lambda i: (0, i, 0)),
            pl.BlockSpec((vocab, d), lambda i: (0, 0)),
            pl.BlockSpec((d, vocab), lambda i: (0, 0)),
            pl.BlockSpec((sblk, d), lambda i: (i, 0)),
            pl.BlockSpec((d,), lambda i: (0,)),
            pl.BlockSpec((d,), lambda i: (0,)),
        ],
        out_specs=pl.BlockSpec((b, sblk, d), lambda i: (0, i, 0)),
        out_shape=jax.ShapeDtypeStruct((b, s, d), jnp.float32),
    )(ids, tok_emb, tok_t, pos, gamma, beta)
    return out
